# baseline (device time: 13078 ns/iter reference)
import jax
import jax.numpy as jnp
from jax import lax
from jax.experimental import pallas as pl
from jax.experimental.pallas import tpu as pltpu

C = 8


def kernel(x):
    _, m, n = x.shape
    h = m // 2
    ch = h // C

    def body(x_ref, out_ref, buf1_ref, buf2_ref,
             send1, recv1, send2, recv2):
        my = lax.axis_index("i")
        p1 = 3 - my
        p2 = my ^ 1
        partner = {1: (p1, p2), 2: (p2, p1)}

        def rows(hf, c):
            return pl.ds(hf * h + c * ch, ch)

        def rdma(stage, hf, c):
            buf, snd, rcv = ((buf1_ref, send1, recv1) if stage == 1
                             else (buf2_ref, send2, recv2))
            return pltpu.make_async_remote_copy(
                src_ref=out_ref.at[rows(hf, c)],
                dst_ref=buf.at[hf, c],
                send_sem=snd.at[hf, c],
                recv_sem=rcv.at[hf, c],
                device_id=(partner[stage][hf],),
                device_id_type=pl.DeviceIdType.MESH,
            )

        barrier_sem = pltpu.get_barrier_semaphore()
        for nbr in [p1, p2]:
            pl.semaphore_signal(
                barrier_sem, inc=1,
                device_id=(nbr,), device_id_type=pl.DeviceIdType.MESH,
            )
        for c in range(C):
            for hf in (0, 1):
                r = rows(hf, c)
                out_ref[r] = x_ref[0, r, :].astype(jnp.bfloat16)
        pl.semaphore_wait(barrier_sem, 2)

        for c in range(C):
            for hf in (0, 1):
                rdma(1, hf, c).start()

        for c in range(C):
            for hf in (0, 1):
                rdma(1, hf, c).wait()
                r = rows(hf, c)
                out_ref[r] = out_ref[r] + buf1_ref[hf, c]
                rdma(2, hf, c).start()

        for c in range(C):
            for hf in (0, 1):
                rdma(2, hf, c).wait()
                r = rows(hf, c)
                out_ref[r] = out_ref[r] + buf2_ref[hf, c]

    return pl.pallas_call(
        body,
        out_shape=jax.ShapeDtypeStruct((m, n), jnp.bfloat16),
        in_specs=[pl.BlockSpec(memory_space=pltpu.VMEM)],
        out_specs=pl.BlockSpec(memory_space=pltpu.VMEM),
        scratch_shapes=[
            pltpu.VMEM((2, C, ch, n), jnp.bfloat16),
            pltpu.VMEM((2, C, ch, n), jnp.bfloat16),
            pltpu.SemaphoreType.DMA((2, C)),
            pltpu.SemaphoreType.DMA((2, C)),
            pltpu.SemaphoreType.DMA((2, C)),
            pltpu.SemaphoreType.DMA((2, C)),
        ],
        compiler_params=pltpu.CompilerParams(collective_id=0),
    )(x)


# device time: 12233 ns/iter; 1.0691x vs baseline; 1.0691x over previous
import jax
import jax.numpy as jnp
from jax import lax
from jax.experimental import pallas as pl
from jax.experimental.pallas import tpu as pltpu

C = 2


def kernel(x):
    _, m, n = x.shape
    h = m // 2
    ch = h // C

    def body(x_ref, out_ref, buf1_ref, buf2_ref,
             send1, recv1, send2, recv2):
        my = lax.axis_index("i")
        p1 = 3 - my
        p2 = my ^ 1
        partner = {1: (p1, p2), 2: (p2, p1)}

        def rows(hf, c):
            return pl.ds(hf * h + c * ch, ch)

        def rdma(stage, hf, c):
            buf, snd, rcv = ((buf1_ref, send1, recv1) if stage == 1
                             else (buf2_ref, send2, recv2))
            return pltpu.make_async_remote_copy(
                src_ref=out_ref.at[rows(hf, c)],
                dst_ref=buf.at[hf, c],
                send_sem=snd.at[hf, c],
                recv_sem=rcv.at[hf, c],
                device_id=(partner[stage][hf],),
                device_id_type=pl.DeviceIdType.MESH,
            )

        barrier_sem = pltpu.get_barrier_semaphore()
        for nbr in [p1, p2]:
            pl.semaphore_signal(
                barrier_sem, inc=1,
                device_id=(nbr,), device_id_type=pl.DeviceIdType.MESH,
            )
        for c in range(C):
            for hf in (0, 1):
                r = rows(hf, c)
                out_ref[r] = x_ref[0, r, :].astype(jnp.bfloat16)
        pl.semaphore_wait(barrier_sem, 2)

        for c in range(C):
            for hf in (0, 1):
                rdma(1, hf, c).start()

        for c in range(C):
            for hf in (0, 1):
                rdma(1, hf, c).wait()
                r = rows(hf, c)
                out_ref[r] = out_ref[r] + buf1_ref[hf, c]
                rdma(2, hf, c).start()

        for c in range(C):
            for hf in (0, 1):
                rdma(2, hf, c).wait()
                r = rows(hf, c)
                out_ref[r] = out_ref[r] + buf2_ref[hf, c]

    return pl.pallas_call(
        body,
        out_shape=jax.ShapeDtypeStruct((m, n), jnp.bfloat16),
        in_specs=[pl.BlockSpec(memory_space=pltpu.VMEM)],
        out_specs=pl.BlockSpec(memory_space=pltpu.VMEM),
        scratch_shapes=[
            pltpu.VMEM((2, C, ch, n), jnp.bfloat16),
            pltpu.VMEM((2, C, ch, n), jnp.bfloat16),
            pltpu.SemaphoreType.DMA((2, C)),
            pltpu.SemaphoreType.DMA((2, C)),
            pltpu.SemaphoreType.DMA((2, C)),
            pltpu.SemaphoreType.DMA((2, C)),
        ],
        compiler_params=pltpu.CompilerParams(collective_id=0),
    )(x)
